# baseline (speedup 1.0000x reference)
import functools

import jax
import jax.numpy as jnp
from jax import lax
from jax.experimental import pallas as pl
from jax.experimental.pallas import tpu as pltpu
from jax.experimental.pallas import tpu_sc as plsc

L = 50
B = 1024
EMBED = 64
HIDDEN = 128


def _make_sc_gather(embed, n_ids):
    info = plsc.get_sparse_core_info()
    nc, ns = info.num_cores, info.num_subcores
    nw = nc * ns
    assert n_ids % nw == 0
    b_per_w = n_ids // nw
    assert b_per_w % 8 == 0

    mesh = plsc.VectorSubcoreMesh(core_axis_name="c", subcore_axis_name="s")

    @functools.partial(
        pl.kernel,
        mesh=mesh,
        out_type=jax.ShapeDtypeStruct((n_ids, embed), jnp.float32),
        scratch_types=[
            pltpu.VMEM((b_per_w,), jnp.int32),
            pltpu.VMEM((b_per_w, embed), jnp.float32),
            pltpu.SemaphoreType.DMA,
        ],
        compiler_params=pltpu.CompilerParams(use_tc_tiling_on_sc=False),
    )
    def gather(table_hbm, idx_hbm, out_hbm, idx_v, rows_v, sem):
        wid = lax.axis_index("s") * nc + lax.axis_index("c")
        base = wid * b_per_w
        pltpu.sync_copy(idx_hbm.at[pl.ds(base, b_per_w)], idx_v)
        pltpu.async_copy(table_hbm.at[idx_v], rows_v, sem).wait()
        pltpu.sync_copy(rows_v, out_hbm.at[pl.ds(base, b_per_w)])

    return gather


def _gru_tc_body(emb_ref, wih_f, whh_f, bih_f, bhh_f, wih_b, whh_b, bih_b,
                 bhh_b, out_ref, xpf_buf, xpb_buf):
    Bb = emb_ref.shape[1]
    H = HIDDEN

    wihf = wih_f[...]
    whhf = whh_f[...]
    wihb = wih_b[...]
    whhb = whh_b[...]
    bihf = bih_f[...]
    bhhf = bhh_f[...]
    bihb = bih_b[...]
    bhhb = bhh_b[...]

    def sig(v):
        return 0.5 * jnp.tanh(0.5 * v) + 0.5

    def gru_step(xp, h, whh, bhh):
        gh = jnp.dot(h, whh, preferred_element_type=jnp.float32) + bhh
        r = sig(xp[:, 0:H] + gh[:, 0:H])
        z = sig(xp[:, H:2 * H] + gh[:, H:2 * H])
        n = jnp.tanh(xp[:, 2 * H:3 * H] + r * gh[:, 2 * H:3 * H])
        return (1.0 - z) * n + z * h

    def xproj(t, wih, bih):
        return jnp.dot(emb_ref[t], wih, preferred_element_type=jnp.float32) + bih

    xpf_buf[0] = xproj(0, wihf, bihf)
    xpb_buf[0] = xproj(L - 1, wihb, bihb)

    def step(t, carry):
        h_f, h_b = carry
        slot = lax.rem(t, 2)
        xpf = xpf_buf[slot]
        xpb = xpb_buf[slot]
        h_f = gru_step(xpf, h_f, whhf, bhhf)
        out_ref[t, :, 0:H] = h_f
        h_b = gru_step(xpb, h_b, whhb, bhhb)
        out_ref[L - 1 - t, :, H:2 * H] = h_b
        tn = jnp.minimum(t + 1, L - 1)
        xpf_buf[1 - slot] = xproj(tn, wihf, bihf)
        xpb_buf[1 - slot] = xproj(L - 1 - tn, wihb, bihb)
        return h_f, h_b

    h0 = jnp.zeros((Bb, H), jnp.float32)
    lax.fori_loop(0, L, step, (h0, h0), unroll=False)


def _make_tc_gru(bb):
    grid = (B // bb,)
    full = lambda i: (0, 0)
    return pl.pallas_call(
        _gru_tc_body,
        grid=grid,
        in_specs=[
            pl.BlockSpec((L, bb, EMBED), lambda i: (0, i, 0)),
            pl.BlockSpec((EMBED, 3 * HIDDEN), full),
            pl.BlockSpec((HIDDEN, 3 * HIDDEN), full),
            pl.BlockSpec((1, 3 * HIDDEN), full),
            pl.BlockSpec((1, 3 * HIDDEN), full),
            pl.BlockSpec((EMBED, 3 * HIDDEN), full),
            pl.BlockSpec((HIDDEN, 3 * HIDDEN), full),
            pl.BlockSpec((1, 3 * HIDDEN), full),
            pl.BlockSpec((1, 3 * HIDDEN), full),
        ],
        out_specs=pl.BlockSpec((L, bb, 2 * HIDDEN), lambda i: (0, i, 0)),
        out_shape=jax.ShapeDtypeStruct((L, B, 2 * HIDDEN), jnp.float32),
        scratch_shapes=[
            pltpu.VMEM((2, bb, 3 * HIDDEN), jnp.float32),
            pltpu.VMEM((2, bb, 3 * HIDDEN), jnp.float32),
        ],
        compiler_params=pltpu.CompilerParams(
            dimension_semantics=("arbitrary",),
        ),
    )


_BB = 256


def kernel(input_seq, input_len, emb_table, W_ih_f, W_hh_f, b_ih_f, b_hh_f,
           W_ih_b, W_hh_b, b_ih_b, b_hh_b):
    del input_len
    embed = emb_table.shape[1]
    ids = input_seq.reshape(-1).astype(jnp.int32)

    rows = _make_sc_gather(embed, ids.shape[0])(emb_table, ids)
    emb = rows.reshape(L, B, embed)

    out = _make_tc_gru(_BB)(
        emb,
        W_ih_f.T, W_hh_f.T, b_ih_f[None, :], b_hh_f[None, :],
        W_ih_b.T, W_hh_b.T, b_ih_b[None, :], b_hh_b[None, :])
    return out


# speedup vs baseline: 1.3642x; 1.3642x over previous
import functools

import jax
import jax.numpy as jnp
from jax import lax
from jax.experimental import pallas as pl
from jax.experimental.pallas import tpu as pltpu
from jax.experimental.pallas import tpu_sc as plsc

L = 50
B = 1024
EMBED = 64
HIDDEN = 128


def _make_sc_gather(embed, n_ids):
    info = plsc.get_sparse_core_info()
    nc, ns = info.num_cores, info.num_subcores
    nw = nc * ns
    assert n_ids % nw == 0
    b_per_w = n_ids // nw
    assert b_per_w % 8 == 0

    mesh = plsc.VectorSubcoreMesh(core_axis_name="c", subcore_axis_name="s")

    @functools.partial(
        pl.kernel,
        mesh=mesh,
        out_type=jax.ShapeDtypeStruct((n_ids, 2 * embed), jnp.float32),
        scratch_types=[
            pltpu.VMEM((b_per_w,), jnp.int32),
            pltpu.VMEM((b_per_w, embed), jnp.float32),
            pltpu.SemaphoreType.DMA,
        ],
        compiler_params=pltpu.CompilerParams(use_tc_tiling_on_sc=False),
    )
    def gather(table_hbm, idx_hbm, out_hbm, idx_v, rows_v, sem):
        wid = lax.axis_index("s") * nc + lax.axis_index("c")
        base = wid * b_per_w
        pltpu.sync_copy(idx_hbm.at[pl.ds(base, b_per_w)], idx_v)
        pltpu.async_copy(table_hbm.at[idx_v], rows_v, sem).wait()
        pltpu.sync_copy(rows_v, out_hbm.at[pl.ds(base, b_per_w), 0:embed])

    return gather


def _gru_tc_body(emb_ref, wih_f, whh_f, bih_f, bhh_f, wih_b, whh_b, bih_b,
                 bhh_b, out_ref):
    Bb = emb_ref.shape[1]
    H = HIDDEN

    wihf = wih_f[...]
    whhf = whh_f[...]
    wihb = wih_b[...]
    whhb = whh_b[...]
    bihf = bih_f[...]
    bhhf = bhh_f[...]
    bihb = bih_b[...]
    bhhb = bhh_b[...]

    def sig(v):
        return 0.5 * jnp.tanh(0.5 * v) + 0.5

    def gru_step(xp, h, whh, bhh):
        gh = jnp.dot(h.astype(jnp.bfloat16), whh,
                     preferred_element_type=jnp.float32) + bhh
        r = sig(xp[:, 0:H] + gh[:, 0:H])
        z = sig(xp[:, H:2 * H] + gh[:, H:2 * H])
        n = jnp.tanh(xp[:, 2 * H:3 * H] + r * gh[:, 2 * H:3 * H])
        return (1.0 - z) * n + z * h

    def xproj(t, wih, bih):
        return jnp.dot(emb_ref[t].astype(jnp.bfloat16), wih,
                       preferred_element_type=jnp.float32) + bih

    def step(t, carry):
        h_f, h_b = carry
        h_f = gru_step(xproj(t, wihf, bihf), h_f, whhf, bhhf)
        out_ref[t, :, 0:H] = h_f
        tb = L - 1 - t
        h_b = gru_step(xproj(tb, wihb, bihb), h_b, whhb, bhhb)
        out_ref[tb, :, H:2 * H] = h_b
        return h_f, h_b

    h0 = jnp.zeros((Bb, H), jnp.float32)
    lax.fori_loop(0, L, step, (h0, h0), unroll=2)


def _make_tc_gru(bb):
    grid = (B // bb,)
    full = lambda i: (0, 0)
    return pl.pallas_call(
        _gru_tc_body,
        grid=grid,
        in_specs=[
            pl.BlockSpec((L, bb, 2 * EMBED), lambda i: (0, i, 0)),
            pl.BlockSpec((2 * EMBED, 3 * HIDDEN), full),
            pl.BlockSpec((HIDDEN, 3 * HIDDEN), full),
            pl.BlockSpec((1, 3 * HIDDEN), full),
            pl.BlockSpec((1, 3 * HIDDEN), full),
            pl.BlockSpec((2 * EMBED, 3 * HIDDEN), full),
            pl.BlockSpec((HIDDEN, 3 * HIDDEN), full),
            pl.BlockSpec((1, 3 * HIDDEN), full),
            pl.BlockSpec((1, 3 * HIDDEN), full),
        ],
        out_specs=pl.BlockSpec((L, bb, 2 * HIDDEN), lambda i: (0, i, 0)),
        out_shape=jax.ShapeDtypeStruct((L, B, 2 * HIDDEN), jnp.float32),
        compiler_params=pltpu.CompilerParams(
            dimension_semantics=("arbitrary",),
        ),
    )


_BB = 256


def _gru_stream_body(emb_hbm, wih_f, whh_f, bih_f, bhh_f, wih_b, whh_b,
                     bih_b, bhh_b, out_hbm, xbuf, obuf, isem, osem):
    H = HIDDEN

    wihf = wih_f[...]
    whhf = whh_f[...]
    wihb = wih_b[...]
    whhb = whh_b[...]
    bihf = bih_f[...]
    bhhf = bhh_f[...]
    bihb = bih_b[...]
    bhhb = bhh_b[...]

    def in_copy(slot, d, t):
        return pltpu.make_async_copy(
            emb_hbm.at[t], xbuf.at[slot, d], isem.at[slot, d])

    def out_copy(slot, d, t):
        return pltpu.make_async_copy(
            obuf.at[slot, d], out_hbm.at[t, :, pl.ds(d * H, H)],
            osem.at[slot, d])

    def sig(v):
        return 0.5 * jnp.tanh(0.5 * v) + 0.5

    def gru_step(x, h, wih, whh, bih, bhh):
        xp = jnp.dot(x[:, 0:EMBED].astype(jnp.bfloat16), wih,
                     preferred_element_type=jnp.float32) + bih
        gh = jnp.dot(h.astype(jnp.bfloat16), whh,
                     preferred_element_type=jnp.float32) + bhh
        r = sig(xp[:, 0:H] + gh[:, 0:H])
        z = sig(xp[:, H:2 * H] + gh[:, H:2 * H])
        n = jnp.tanh(xp[:, 2 * H:3 * H] + r * gh[:, 2 * H:3 * H])
        return (1.0 - z) * n + z * h

    in_copy(0, 0, 0).start()
    in_copy(0, 1, L - 1).start()

    def step(t, carry):
        h_f, h_b = carry
        slot = lax.rem(t, 2)
        nslot = 1 - slot

        @pl.when(t < L - 1)
        def _():
            in_copy(nslot, 0, t + 1).start()
            in_copy(nslot, 1, L - 2 - t).start()

        in_copy(slot, 0, t).wait()
        in_copy(slot, 1, L - 1 - t).wait()

        @pl.when(t >= 2)
        def _():
            out_copy(slot, 0, t - 2).wait()
            out_copy(slot, 1, L + 1 - t).wait()

        h_f = gru_step(xbuf[slot, 0], h_f, wihf, whhf, bihf, bhhf)
        h_b = gru_step(xbuf[slot, 1], h_b, wihb, whhb, bihb, bhhb)
        obuf[slot, 0] = h_f
        obuf[slot, 1] = h_b
        out_copy(slot, 0, t).start()
        out_copy(slot, 1, L - 1 - t).start()
        return h_f, h_b

    h0 = jnp.zeros((B, H), jnp.float32)
    lax.fori_loop(0, L, step, (h0, h0), unroll=2)

    out_copy(0, 0, L - 2).wait()
    out_copy(0, 1, 1).wait()
    out_copy(1, 0, L - 1).wait()
    out_copy(1, 1, 0).wait()


def _make_tc_gru_stream():
    vm = pl.BlockSpec(memory_space=pltpu.VMEM)
    return pl.pallas_call(
        _gru_stream_body,
        in_specs=[pl.BlockSpec(memory_space=pl.ANY),
                  vm, vm, vm, vm, vm, vm, vm, vm],
        out_specs=pl.BlockSpec(memory_space=pl.ANY),
        out_shape=jax.ShapeDtypeStruct((L, B, 2 * HIDDEN), jnp.float32),
        scratch_shapes=[
            pltpu.VMEM((2, 2, B, 2 * EMBED), jnp.float32),
            pltpu.VMEM((2, 2, B, HIDDEN), jnp.float32),
            pltpu.SemaphoreType.DMA((2, 2)),
            pltpu.SemaphoreType.DMA((2, 2)),
        ],
    )


def kernel(input_seq, input_len, emb_table, W_ih_f, W_hh_f, b_ih_f, b_hh_f,
           W_ih_b, W_hh_b, b_ih_b, b_hh_b):
    del input_len
    embed = emb_table.shape[1]
    ids = input_seq.reshape(-1).astype(jnp.int32)

    rows = _make_sc_gather(embed, ids.shape[0])(emb_table, ids)
    emb = rows.reshape(L, B, 2 * embed)

    bf16 = jnp.bfloat16
    out = _make_tc_gru_stream()(
        emb,
        W_ih_f.T.astype(bf16), W_hh_f.T.astype(bf16),
        b_ih_f[None, :], b_hh_f[None, :],
        W_ih_b.T.astype(bf16), W_hh_b.T.astype(bf16),
        b_ih_b[None, :], b_hh_b[None, :])
    return out
